# baseline (speedup 1.0000x reference)
import functools

import jax
import jax.numpy as jnp
from jax import lax
from jax.experimental import pallas as pl
from jax.experimental.pallas import tpu as pltpu
from jax.experimental.pallas import tpu_sc as plsc

B, S, H = 1, 2048, 2048
I = 1024
E = 8
K = 2
P = S * K
T = 128
NT = P // T
NITEMS = NT + E - 1

NC, NS = 2, 16
NW = NC * NS


def _routing_body(x_ref, gw_ref, bias_ref, pos0_ref, pos1_ref, p0_ref, p1_ref,
                  counts_ref, mv_ref):
    xf = x_ref[...]
    scores = lax.dot_general(
        xf, gw_ref[...], (((1,), (1,)), ((), ())),
        preferred_element_type=jnp.float32)
    biased = scores + bias_ref[...]
    iota = lax.broadcasted_iota(jnp.int32, (S, E), 1)
    neg_inf = jnp.float32(-jnp.inf)

    v1 = jnp.max(biased, axis=1, keepdims=True)
    idx1 = jnp.min(jnp.where(biased == v1, iota, E), axis=1, keepdims=True)
    masked = jnp.where(iota == idx1, neg_inf, biased)
    v2 = jnp.max(masked, axis=1, keepdims=True)
    idx2 = jnp.min(jnp.where(masked == v2, iota, E), axis=1, keepdims=True)

    u1 = jnp.max(scores, axis=1, keepdims=True)
    uidx1 = jnp.min(jnp.where(scores == u1, iota, E), axis=1, keepdims=True)
    u2 = jnp.max(jnp.where(iota == uidx1, neg_inf, scores), axis=1,
                 keepdims=True)
    p1 = jax.nn.sigmoid(u1)
    p2 = jax.nn.sigmoid(u2)
    ps = p1 + p2
    p0_ref[...] = p1 / ps
    p1_ref[...] = p2 / ps

    oh1 = (iota == idx1).astype(jnp.float32)
    oh2 = (iota == idx2).astype(jnp.float32)

    counts = jnp.sum(oh1 + oh2, axis=0, keepdims=True)
    counts_ref[...] = counts
    freq = counts / jnp.float32(P)
    fmean = jnp.sum(freq) / jnp.float32(E)
    mv_ref[...] = jnp.full((1, 1), (jnp.max(freq) - fmean) / fmean,
                           jnp.float32)

    tri = (lax.broadcasted_iota(jnp.int32, (S, S), 1)
           < lax.broadcasted_iota(jnp.int32, (S, S), 0)).astype(jnp.bfloat16)
    c1ex = lax.dot_general(tri, oh1.astype(jnp.bfloat16),
                           (((1,), (0,)), ((), ())),
                           preferred_element_type=jnp.float32)
    c2ex = lax.dot_general(tri, oh2.astype(jnp.bfloat16),
                           (((1,), (0,)), ((), ())),
                           preferred_element_type=jnp.float32)
    lt1 = (idx1 < iota).astype(jnp.float32)
    lt2 = (idx2 < iota).astype(jnp.float32)
    offsets = jnp.sum(lt1 + lt2, axis=0, keepdims=True)
    c1tot = jnp.sum(oh1, axis=0, keepdims=True)
    pos0_ref[...] = jnp.sum((offsets + c1ex) * oh1, axis=1, keepdims=True)
    pos1_ref[...] = jnp.sum((offsets + c1tot + c2ex) * oh2, axis=1,
                            keepdims=True)


def _shared_body(x_ref, sg_ref, su_ref, sd_ref, out_ref):
    g = lax.dot_general(x_ref[...], sg_ref[...], (((1,), (1,)), ((), ())),
                        preferred_element_type=jnp.float32)
    u = lax.dot_general(x_ref[...], su_ref[...], (((1,), (1,)), ((), ())),
                        preferred_element_type=jnp.float32)
    h = (jax.nn.silu(g) * u).astype(jnp.bfloat16)
    out_ref[...] = lax.dot_general(h, sd_ref[...], (((1,), (1,)), ((), ())),
                                   preferred_element_type=jnp.float32)


def _grouped_body(tile_s, exp_s, lo_s, hi_s, x_ref, wg_ref, wu_ref, wd_ref,
                  y_ref):
    i = pl.program_id(0)
    lo = lo_s[i]
    hi = hi_s[i]
    tile = tile_s[i]

    @pl.when(hi > lo)
    def _work():
        xb = x_ref[...]
        g = lax.dot_general(xb, wg_ref[0], (((1,), (1,)), ((), ())),
                            preferred_element_type=jnp.float32)
        u = lax.dot_general(xb, wu_ref[0], (((1,), (1,)), ((), ())),
                            preferred_element_type=jnp.float32)
        h = (jax.nn.silu(g) * u).astype(jnp.bfloat16)
        y = lax.dot_general(h, wd_ref[0], (((1,), (1,)), ((), ())),
                            preferred_element_type=jnp.float32)
        rows = tile * T + lax.broadcasted_iota(jnp.int32, (T, 1), 0)
        m = ((rows >= lo) & (rows < hi)).astype(jnp.float32)
        contrib = y * m

        @pl.when(lo == tile * T)
        def _init():
            y_ref[...] = contrib

        @pl.when(lo != tile * T)
        def _acc():
            y_ref[...] += contrib


def _mesh():
    return plsc.VectorSubcoreMesh(core_axis_name="c", subcore_axis_name="s",
                                  num_cores=NC, num_subcores=NS)


@functools.cache
def _build_sc_dispatch():
    return functools.partial(
        pl.kernel,
        out_type=jax.ShapeDtypeStruct((P, H // 2), jnp.int32),
        mesh=_mesh(),
        scratch_types=[
            pltpu.VMEM((32,), jnp.int32),
            pltpu.VMEM((32,), jnp.int32),
            pltpu.VMEM((32, H // 2), jnp.int32),
            pltpu.VMEM((32, H // 2), jnp.int32),
            pltpu.SemaphoreType.DMA,
            pltpu.SemaphoreType.DMA,
            pltpu.SemaphoreType.DMA,
            pltpu.SemaphoreType.DMA,
        ],
    )(_sc_dispatch_body)


def _sc_dispatch(x32, pos):
    return _build_sc_dispatch()(x32, pos)


def _sc_dispatch_body(x_hbm, pos_hbm, xs_hbm, idx_a, idx_b, row_a, row_b,
                      sem_ra, sem_rb, sem_wa, sem_wb):
    wid = lax.axis_index("s") * NC + lax.axis_index("c")
    k = wid // 16
    tb = (wid % 16) * 128
    idx = [idx_a, idx_b]
    row = [row_a, row_b]
    sem_r = [sem_ra, sem_rb]
    sem_w = [sem_wa, sem_wb]
    reads = [None, None]
    writes = [None, None]
    pltpu.sync_copy(pos_hbm.at[k, pl.ds(tb, 32)], idx[0])
    reads[0] = pltpu.async_copy(x_hbm.at[pl.ds(tb, 32)], row[0], sem_r[0])
    for c in range(4):
        s = c % 2
        o = 1 - s
        if c + 1 < 4:
            if writes[o] is not None:
                writes[o].wait()
                writes[o] = None
            base = tb + 32 * (c + 1)
            pltpu.sync_copy(pos_hbm.at[k, pl.ds(base, 32)], idx[o])
            reads[o] = pltpu.async_copy(x_hbm.at[pl.ds(base, 32)], row[o],
                                        sem_r[o])
        reads[s].wait()
        writes[s] = pltpu.async_copy(row[s], xs_hbm.at[idx[s]], sem_w[s])
    writes[0].wait()
    writes[1].wait()


_CT = 8
_NCH = 64 // _CT


@functools.cache
def _build_sc_combine():
    return functools.partial(
        pl.kernel,
        out_type=jax.ShapeDtypeStruct((S, H), jnp.float32),
        mesh=_mesh(),
        scratch_types=[
            pltpu.VMEM((2 * _CT,), jnp.int32),
            pltpu.VMEM((2 * _CT,), jnp.int32),
            pltpu.VMEM((2 * _CT, 16), jnp.float32),
            pltpu.VMEM((2 * _CT, 16), jnp.float32),
            pltpu.VMEM((2 * _CT, H), jnp.float32),
            pltpu.VMEM((2 * _CT, H), jnp.float32),
            pltpu.VMEM((_CT, H), jnp.float32),
            pltpu.VMEM((_CT, H), jnp.float32),
            pltpu.SemaphoreType.DMA,
            pltpu.SemaphoreType.DMA,
            pltpu.SemaphoreType.DMA,
            pltpu.SemaphoreType.DMA,
        ],
    )(_sc_combine_body)


def _sc_combine(shared, y, pos, pw):
    return _build_sc_combine()(shared, y, pos, pw)


def _sc_combine_body(sh_hbm, y_hbm, pos_hbm, pw_hbm, out_hbm,
                     idx_a, idx_b, p_a, p_b, y_a, y_b, s_a, s_b,
                     sem_ga, sem_gb, sem_wa, sem_wb):
    wid = lax.axis_index("s") * NC + lax.axis_index("c")
    idx = [idx_a, idx_b]
    pb = [p_a, p_b]
    yb = [y_a, y_b]
    sb = [s_a, s_b]
    sem_g = [sem_ga, sem_gb]
    sem_w = [sem_wa, sem_wb]
    gets = [None, None]
    puts = [None, None]

    def start(c, s):
        base = wid * 64 + _CT * c
        pltpu.sync_copy(pos_hbm.at[0, pl.ds(base, _CT)],
                        idx[s].at[pl.ds(0, _CT)])
        pltpu.sync_copy(pos_hbm.at[1, pl.ds(base, _CT)],
                        idx[s].at[pl.ds(_CT, _CT)])
        pltpu.sync_copy(pw_hbm.at[0, pl.ds(base, _CT)],
                        pb[s].at[pl.ds(0, _CT)])
        pltpu.sync_copy(pw_hbm.at[1, pl.ds(base, _CT)],
                        pb[s].at[pl.ds(_CT, _CT)])
        gets[s] = (pltpu.async_copy(y_hbm.at[idx[s]], yb[s], sem_g[s]),
                   pltpu.async_copy(sh_hbm.at[pl.ds(base, _CT)], sb[s],
                                    sem_g[s]))

    start(0, 0)
    for c in range(_NCH):
        s = c % 2
        o = 1 - s
        if c + 1 < _NCH:
            if puts[o] is not None:
                puts[o].wait()
                puts[o] = None
            start(c + 1, o)
        gets[s][0].wait()
        gets[s][1].wait()

        def tok(t, _):
            p0b = pb[s][t, :]
            p1b = pb[s][_CT + t, :]

            def col(j, _):
                d = pl.ds(j * 16, 16)
                sb[s][t, d] = (sb[s][t, d] + p0b * yb[s][t, d]
                               + p1b * yb[s][_CT + t, d])
                return 0

            lax.fori_loop(0, H // 16, col, 0, unroll=8)
            return 0

        lax.fori_loop(0, _CT, tok, 0)
        base = wid * 64 + _CT * c
        puts[s] = pltpu.async_copy(sb[s], out_hbm.at[pl.ds(base, _CT)],
                                   sem_w[s])
    puts[0].wait()
    puts[1].wait()


def _make_schedule(counts):
    counts_i = counts[0].astype(jnp.int32)
    offs = jnp.concatenate(
        [jnp.zeros((1,), jnp.int32), jnp.cumsum(counts_i)])
    bp = jnp.sort(jnp.concatenate(
        [jnp.arange(NT, dtype=jnp.int32) * T, offs[1:E]]))
    nxt = jnp.concatenate([bp[1:], jnp.array([P], jnp.int32)])
    item_tile = jnp.clip(bp // T, 0, NT - 1)
    item_expert = jnp.clip(
        jnp.searchsorted(offs, bp, side="right") - 1, 0, E - 1
    ).astype(jnp.int32)
    return item_tile, item_expert, bp, nxt


@jax.jit
def kernel(x, gate_w, e_bias, wg, wu, wd, sg, su, sd):
    xf = x.reshape(S, H)
    x_bf = xf.astype(jnp.bfloat16)

    pos0, pos1, p0, p1, counts, mv = pl.pallas_call(
        _routing_body,
        out_shape=(
            jax.ShapeDtypeStruct((S, 1), jnp.float32),
            jax.ShapeDtypeStruct((S, 1), jnp.float32),
            jax.ShapeDtypeStruct((S, 1), jnp.float32),
            jax.ShapeDtypeStruct((S, 1), jnp.float32),
            jax.ShapeDtypeStruct((1, E), jnp.float32),
            jax.ShapeDtypeStruct((1, 1), jnp.float32),
        ),
    )(xf, gate_w, e_bias.reshape(1, E))

    pos = jnp.stack([pos0[:, 0], pos1[:, 0]]).astype(jnp.int32)
    pw = jnp.broadcast_to(
        jnp.stack([p0[:, 0], p1[:, 0]])[:, :, None], (K, S, 16))
    item_tile, item_expert, item_lo, item_hi = _make_schedule(counts)

    x32 = lax.bitcast_convert_type(x_bf.reshape(S, H // 2, 2), jnp.int32)
    xs32 = _sc_dispatch(x32, pos)
    x_sorted = lax.bitcast_convert_type(xs32, jnp.bfloat16).reshape(P, H)

    shared = pl.pallas_call(
        _shared_body,
        grid=(2,),
        in_specs=[
            pl.BlockSpec((S // 2, H), lambda t: (t, 0)),
            pl.BlockSpec((I, H), lambda t: (0, 0)),
            pl.BlockSpec((I, H), lambda t: (0, 0)),
            pl.BlockSpec((H, I), lambda t: (0, 0)),
        ],
        out_specs=pl.BlockSpec((S // 2, H), lambda t: (t, 0)),
        out_shape=jax.ShapeDtypeStruct((S, H), jnp.float32),
    )(x_bf, sg.astype(jnp.bfloat16), su.astype(jnp.bfloat16),
      sd.astype(jnp.bfloat16))

    wg_bf = wg.astype(jnp.bfloat16)
    wu_bf = wu.astype(jnp.bfloat16)
    wd_bf = wd.astype(jnp.bfloat16)

    y_sorted = pl.pallas_call(
        _grouped_body,
        grid_spec=pltpu.PrefetchScalarGridSpec(
            num_scalar_prefetch=4,
            grid=(NITEMS,),
            in_specs=[
                pl.BlockSpec((T, H), lambda i, ts, es, ls, hs: (ts[i], 0)),
                pl.BlockSpec((1, I, H),
                             lambda i, ts, es, ls, hs: (es[i], 0, 0)),
                pl.BlockSpec((1, I, H),
                             lambda i, ts, es, ls, hs: (es[i], 0, 0)),
                pl.BlockSpec((1, H, I),
                             lambda i, ts, es, ls, hs: (es[i], 0, 0)),
            ],
            out_specs=pl.BlockSpec((T, H),
                                   lambda i, ts, es, ls, hs: (ts[i], 0)),
        ),
        out_shape=jax.ShapeDtypeStruct((P, H), jnp.float32),
    )(item_tile, item_expert, item_lo, item_hi, x_sorted, wg_bf, wu_bf, wd_bf)

    out = _sc_combine(shared, y_sorted, pos, pw)

    return (out.reshape(B, S, H), jnp.float32(0.0), mv[0, 0])


# speedup vs baseline: 1.4524x; 1.4524x over previous
import functools

import jax
import jax.numpy as jnp
from jax import lax
from jax.experimental import pallas as pl
from jax.experimental.pallas import tpu as pltpu
from jax.experimental.pallas import tpu_sc as plsc

B, S, H = 1, 2048, 2048
I = 1024
E = 8
K = 2
P = S * K
T = 128
NT = P // T
NITEMS = NT + E - 1

NC, NS = 2, 16
NW = NC * NS


def _routing_body(x_ref, gw_ref, bias_ref, pos0_ref, pos1_ref, p0_ref, p1_ref,
                  counts_ref, mv_ref):
    xf = x_ref[...]
    scores = lax.dot_general(
        xf, gw_ref[...], (((1,), (1,)), ((), ())),
        preferred_element_type=jnp.float32)
    biased = scores + bias_ref[...]
    iota = lax.broadcasted_iota(jnp.int32, (S, E), 1)
    neg_inf = jnp.float32(-jnp.inf)

    v1 = jnp.max(biased, axis=1, keepdims=True)
    idx1 = jnp.min(jnp.where(biased == v1, iota, E), axis=1, keepdims=True)
    masked = jnp.where(iota == idx1, neg_inf, biased)
    v2 = jnp.max(masked, axis=1, keepdims=True)
    idx2 = jnp.min(jnp.where(masked == v2, iota, E), axis=1, keepdims=True)

    u1 = jnp.max(scores, axis=1, keepdims=True)
    uidx1 = jnp.min(jnp.where(scores == u1, iota, E), axis=1, keepdims=True)
    u2 = jnp.max(jnp.where(iota == uidx1, neg_inf, scores), axis=1,
                 keepdims=True)
    p1 = jax.nn.sigmoid(u1)
    p2 = jax.nn.sigmoid(u2)
    ps = p1 + p2
    p0_ref[...] = p1 / ps
    p1_ref[...] = p2 / ps

    oh1 = (iota == idx1).astype(jnp.float32)
    oh2 = (iota == idx2).astype(jnp.float32)

    counts = jnp.sum(oh1 + oh2, axis=0, keepdims=True)
    counts_ref[...] = counts
    freq = counts / jnp.float32(P)
    fmean = jnp.sum(freq) / jnp.float32(E)
    mv_ref[...] = jnp.full((1, 1), (jnp.max(freq) - fmean) / fmean,
                           jnp.float32)

    tri = (lax.broadcasted_iota(jnp.int32, (S, S), 1)
           < lax.broadcasted_iota(jnp.int32, (S, S), 0)).astype(jnp.bfloat16)
    c1ex = lax.dot_general(tri, oh1.astype(jnp.bfloat16),
                           (((1,), (0,)), ((), ())),
                           preferred_element_type=jnp.float32)
    c2ex = lax.dot_general(tri, oh2.astype(jnp.bfloat16),
                           (((1,), (0,)), ((), ())),
                           preferred_element_type=jnp.float32)
    lt1 = (idx1 < iota).astype(jnp.float32)
    lt2 = (idx2 < iota).astype(jnp.float32)
    offsets = jnp.sum(lt1 + lt2, axis=0, keepdims=True)
    c1tot = jnp.sum(oh1, axis=0, keepdims=True)
    pos0_ref[...] = jnp.sum((offsets + c1ex) * oh1, axis=1, keepdims=True)
    pos1_ref[...] = jnp.sum((offsets + c1tot + c2ex) * oh2, axis=1,
                            keepdims=True)


def _shared_body(x_ref, sg_ref, su_ref, sd_ref, out_ref):
    g = lax.dot_general(x_ref[...], sg_ref[...], (((1,), (1,)), ((), ())),
                        preferred_element_type=jnp.float32)
    u = lax.dot_general(x_ref[...], su_ref[...], (((1,), (1,)), ((), ())),
                        preferred_element_type=jnp.float32)
    h = (jax.nn.silu(g) * u).astype(jnp.bfloat16)
    out_ref[...] = lax.dot_general(h, sd_ref[...], (((1,), (1,)), ((), ())),
                                   preferred_element_type=jnp.float32)


def _grouped_body(tile_s, exp_s, lo_s, hi_s, x_ref, wg_ref, wu_ref, wd_ref,
                  y_ref):
    i = pl.program_id(0)
    lo = lo_s[i]
    hi = hi_s[i]
    tile = tile_s[i]

    @pl.when(hi > lo)
    def _work():
        xb = x_ref[...].astype(jnp.bfloat16)
        g = lax.dot_general(xb, wg_ref[0], (((1,), (1,)), ((), ())),
                            preferred_element_type=jnp.float32)
        u = lax.dot_general(xb, wu_ref[0], (((1,), (1,)), ((), ())),
                            preferred_element_type=jnp.float32)
        h = (jax.nn.silu(g) * u).astype(jnp.bfloat16)
        y = lax.dot_general(h, wd_ref[0], (((1,), (1,)), ((), ())),
                            preferred_element_type=jnp.float32)
        rows = tile * T + lax.broadcasted_iota(jnp.int32, (T, 1), 0)
        m = ((rows >= lo) & (rows < hi)).astype(jnp.float32)
        contrib = y * m

        @pl.when(lo == tile * T)
        def _init():
            y_ref[...] = contrib

        @pl.when(lo != tile * T)
        def _acc():
            y_ref[...] += contrib


def _mesh():
    return plsc.VectorSubcoreMesh(core_axis_name="c", subcore_axis_name="s",
                                  num_cores=NC, num_subcores=NS)


@functools.cache
def _build_sc_dispatch():
    return functools.partial(
        pl.kernel,
        out_type=jax.ShapeDtypeStruct((P, H), jnp.float32),
        mesh=_mesh(),
        scratch_types=[
            pltpu.VMEM((16,), jnp.int32),
            pltpu.VMEM((16,), jnp.int32),
            pltpu.VMEM((16, H), jnp.float32),
            pltpu.VMEM((16, H), jnp.float32),
            pltpu.SemaphoreType.DMA,
            pltpu.SemaphoreType.DMA,
            pltpu.SemaphoreType.DMA,
            pltpu.SemaphoreType.DMA,
        ],
    )(_sc_dispatch_body)


def _sc_dispatch(xf, pos):
    return _build_sc_dispatch()(xf, pos)


def _sc_dispatch_body(x_hbm, pos_hbm, xs_hbm, idx_a, idx_b, row_a, row_b,
                      sem_ra, sem_rb, sem_wa, sem_wb):
    wid = lax.axis_index("s") * NC + lax.axis_index("c")
    k = wid // 16
    tb = (wid % 16) * 128
    idx = [idx_a, idx_b]
    row = [row_a, row_b]
    sem_r = [sem_ra, sem_rb]
    sem_w = [sem_wa, sem_wb]
    reads = [None, None]
    writes = [None, None]
    nch = 8
    pltpu.sync_copy(pos_hbm.at[k, pl.ds(tb, 16)], idx[0])
    reads[0] = pltpu.async_copy(x_hbm.at[pl.ds(tb, 16)], row[0], sem_r[0])
    for c in range(nch):
        s = c % 2
        o = 1 - s
        if c + 1 < nch:
            if writes[o] is not None:
                writes[o].wait()
                writes[o] = None
            base = tb + 16 * (c + 1)
            pltpu.sync_copy(pos_hbm.at[k, pl.ds(base, 16)], idx[o])
            reads[o] = pltpu.async_copy(x_hbm.at[pl.ds(base, 16)], row[o],
                                        sem_r[o])
        reads[s].wait()
        writes[s] = pltpu.async_copy(row[s], xs_hbm.at[idx[s]], sem_w[s])
    writes[0].wait()
    writes[1].wait()


_CT = 8
_NCH = 64 // _CT


@functools.cache
def _build_sc_combine():
    return functools.partial(
        pl.kernel,
        out_type=jax.ShapeDtypeStruct((S, H), jnp.float32),
        mesh=_mesh(),
        scratch_types=[
            pltpu.VMEM((2 * _CT,), jnp.int32),
            pltpu.VMEM((2 * _CT,), jnp.int32),
            pltpu.VMEM((2 * _CT, 16), jnp.float32),
            pltpu.VMEM((2 * _CT, 16), jnp.float32),
            pltpu.VMEM((2 * _CT, H), jnp.float32),
            pltpu.VMEM((2 * _CT, H), jnp.float32),
            pltpu.VMEM((_CT, H), jnp.float32),
            pltpu.VMEM((_CT, H), jnp.float32),
            pltpu.SemaphoreType.DMA,
            pltpu.SemaphoreType.DMA,
            pltpu.SemaphoreType.DMA,
            pltpu.SemaphoreType.DMA,
        ],
    )(_sc_combine_body)


def _sc_combine(shared, y, pos, pw):
    return _build_sc_combine()(shared, y, pos, pw)


def _sc_combine_body(sh_hbm, y_hbm, pos_hbm, pw_hbm, out_hbm,
                     idx_a, idx_b, p_a, p_b, y_a, y_b, s_a, s_b,
                     sem_ga, sem_gb, sem_wa, sem_wb):
    wid = lax.axis_index("s") * NC + lax.axis_index("c")
    idx = [idx_a, idx_b]
    pb = [p_a, p_b]
    yb = [y_a, y_b]
    sb = [s_a, s_b]
    sem_g = [sem_ga, sem_gb]
    sem_w = [sem_wa, sem_wb]
    gets = [None, None]
    puts = [None, None]

    def start(c, s):
        base = wid * 64 + _CT * c
        pltpu.sync_copy(pos_hbm.at[0, pl.ds(base, _CT)],
                        idx[s].at[pl.ds(0, _CT)])
        pltpu.sync_copy(pos_hbm.at[1, pl.ds(base, _CT)],
                        idx[s].at[pl.ds(_CT, _CT)])
        pltpu.sync_copy(pw_hbm.at[0, pl.ds(base, _CT)],
                        pb[s].at[pl.ds(0, _CT)])
        pltpu.sync_copy(pw_hbm.at[1, pl.ds(base, _CT)],
                        pb[s].at[pl.ds(_CT, _CT)])
        gets[s] = (pltpu.async_copy(y_hbm.at[idx[s]], yb[s], sem_g[s]),
                   pltpu.async_copy(sh_hbm.at[pl.ds(base, _CT)], sb[s],
                                    sem_g[s]))

    start(0, 0)
    for c in range(_NCH):
        s = c % 2
        o = 1 - s
        if c + 1 < _NCH:
            if puts[o] is not None:
                puts[o].wait()
                puts[o] = None
            start(c + 1, o)
        gets[s][0].wait()
        gets[s][1].wait()

        def tok(t, _):
            p0b = pb[s][t, :]
            p1b = pb[s][_CT + t, :]

            def col(j, _):
                d = pl.ds(j * 16, 16)
                sb[s][t, d] = (sb[s][t, d] + p0b * yb[s][t, d]
                               + p1b * yb[s][_CT + t, d])
                return 0

            lax.fori_loop(0, H // 16, col, 0, unroll=8)
            return 0

        lax.fori_loop(0, _CT, tok, 0)
        base = wid * 64 + _CT * c
        puts[s] = pltpu.async_copy(sb[s], out_hbm.at[pl.ds(base, _CT)],
                                   sem_w[s])
    puts[0].wait()
    puts[1].wait()


def _make_schedule(counts):
    counts_i = counts[0].astype(jnp.int32)
    offs = jnp.concatenate(
        [jnp.zeros((1,), jnp.int32), jnp.cumsum(counts_i)])
    bp = jnp.sort(jnp.concatenate(
        [jnp.arange(NT, dtype=jnp.int32) * T, offs[1:E]]))
    nxt = jnp.concatenate([bp[1:], jnp.array([P], jnp.int32)])
    item_tile = jnp.clip(bp // T, 0, NT - 1)
    item_expert = jnp.clip(
        jnp.searchsorted(offs, bp, side="right") - 1, 0, E - 1
    ).astype(jnp.int32)
    return item_tile, item_expert, bp, nxt


@jax.jit
def kernel(x, gate_w, e_bias, wg, wu, wd, sg, su, sd):
    xf = x.reshape(S, H)
    x_bf = xf.astype(jnp.bfloat16)

    pos0, pos1, p0, p1, counts, mv = pl.pallas_call(
        _routing_body,
        out_shape=(
            jax.ShapeDtypeStruct((S, 1), jnp.float32),
            jax.ShapeDtypeStruct((S, 1), jnp.float32),
            jax.ShapeDtypeStruct((S, 1), jnp.float32),
            jax.ShapeDtypeStruct((S, 1), jnp.float32),
            jax.ShapeDtypeStruct((1, E), jnp.float32),
            jax.ShapeDtypeStruct((1, 1), jnp.float32),
        ),
    )(xf, gate_w, e_bias.reshape(1, E))

    pos = jnp.stack([pos0[:, 0], pos1[:, 0]]).astype(jnp.int32)
    pw = jnp.broadcast_to(
        jnp.stack([p0[:, 0], p1[:, 0]])[:, :, None], (K, S, 16))
    item_tile, item_expert, item_lo, item_hi = _make_schedule(counts)

    x_sorted = _sc_dispatch(xf, pos)

    shared = pl.pallas_call(
        _shared_body,
        grid=(2,),
        in_specs=[
            pl.BlockSpec((S // 2, H), lambda t: (t, 0)),
            pl.BlockSpec((I, H), lambda t: (0, 0)),
            pl.BlockSpec((I, H), lambda t: (0, 0)),
            pl.BlockSpec((H, I), lambda t: (0, 0)),
        ],
        out_specs=pl.BlockSpec((S // 2, H), lambda t: (t, 0)),
        out_shape=jax.ShapeDtypeStruct((S, H), jnp.float32),
    )(x_bf, sg.astype(jnp.bfloat16), su.astype(jnp.bfloat16),
      sd.astype(jnp.bfloat16))

    wg_bf = wg.astype(jnp.bfloat16)
    wu_bf = wu.astype(jnp.bfloat16)
    wd_bf = wd.astype(jnp.bfloat16)

    y_sorted = pl.pallas_call(
        _grouped_body,
        grid_spec=pltpu.PrefetchScalarGridSpec(
            num_scalar_prefetch=4,
            grid=(NITEMS,),
            in_specs=[
                pl.BlockSpec((T, H), lambda i, ts, es, ls, hs: (ts[i], 0)),
                pl.BlockSpec((1, I, H),
                             lambda i, ts, es, ls, hs: (es[i], 0, 0)),
                pl.BlockSpec((1, I, H),
                             lambda i, ts, es, ls, hs: (es[i], 0, 0)),
                pl.BlockSpec((1, H, I),
                             lambda i, ts, es, ls, hs: (es[i], 0, 0)),
            ],
            out_specs=pl.BlockSpec((T, H),
                                   lambda i, ts, es, ls, hs: (ts[i], 0)),
        ),
        out_shape=jax.ShapeDtypeStruct((P, H), jnp.float32),
    )(item_tile, item_expert, item_lo, item_hi, x_sorted, wg_bf, wu_bf, wd_bf)

    out = _sc_combine(shared, y_sorted, pos, pw)

    return (out.reshape(B, S, H), jnp.float32(0.0), mv[0, 0])


# speedup vs baseline: 1.8218x; 1.2544x over previous
import functools

import jax
import jax.numpy as jnp
from jax import lax
from jax.experimental import pallas as pl
from jax.experimental.pallas import tpu as pltpu
from jax.experimental.pallas import tpu_sc as plsc

B, S, H = 1, 2048, 2048
I = 1024
E = 8
K = 2
P = S * K
T = 128
NT = P // T
NITEMS = NT + E - 1

NC, NS = 2, 16
NW = NC * NS


def _routing_body(x_ref, gw_ref, bias_ref, pos0_ref, pos1_ref, p0_ref, p1_ref,
                  counts_ref, mv_ref):
    xf = x_ref[...]
    scores = lax.dot_general(
        xf, gw_ref[...], (((1,), (1,)), ((), ())),
        preferred_element_type=jnp.float32)
    biased = scores + bias_ref[...]
    iota = lax.broadcasted_iota(jnp.int32, (S, E), 1)
    neg_inf = jnp.float32(-jnp.inf)

    v1 = jnp.max(biased, axis=1, keepdims=True)
    idx1 = jnp.min(jnp.where(biased == v1, iota, E), axis=1, keepdims=True)
    masked = jnp.where(iota == idx1, neg_inf, biased)
    v2 = jnp.max(masked, axis=1, keepdims=True)
    idx2 = jnp.min(jnp.where(masked == v2, iota, E), axis=1, keepdims=True)

    u1 = jnp.max(scores, axis=1, keepdims=True)
    uidx1 = jnp.min(jnp.where(scores == u1, iota, E), axis=1, keepdims=True)
    u2 = jnp.max(jnp.where(iota == uidx1, neg_inf, scores), axis=1,
                 keepdims=True)
    p1 = jax.nn.sigmoid(u1)
    p2 = jax.nn.sigmoid(u2)
    ps = p1 + p2
    p0_ref[...] = p1 / ps
    p1_ref[...] = p2 / ps

    oh1 = (iota == idx1).astype(jnp.float32)
    oh2 = (iota == idx2).astype(jnp.float32)

    counts = jnp.sum(oh1 + oh2, axis=0, keepdims=True)
    counts_ref[...] = counts
    freq = counts / jnp.float32(P)
    fmean = jnp.sum(freq) / jnp.float32(E)
    mv_ref[...] = jnp.full((1, 1), (jnp.max(freq) - fmean) / fmean,
                           jnp.float32)

    tri = (lax.broadcasted_iota(jnp.int32, (S, S), 1)
           < lax.broadcasted_iota(jnp.int32, (S, S), 0)).astype(jnp.bfloat16)
    c1ex = lax.dot_general(tri, oh1.astype(jnp.bfloat16),
                           (((1,), (0,)), ((), ())),
                           preferred_element_type=jnp.float32)
    c2ex = lax.dot_general(tri, oh2.astype(jnp.bfloat16),
                           (((1,), (0,)), ((), ())),
                           preferred_element_type=jnp.float32)
    lt1 = (idx1 < iota).astype(jnp.float32)
    lt2 = (idx2 < iota).astype(jnp.float32)
    offsets = jnp.sum(lt1 + lt2, axis=0, keepdims=True)
    c1tot = jnp.sum(oh1, axis=0, keepdims=True)
    pos0_ref[...] = jnp.sum((offsets + c1ex) * oh1, axis=1, keepdims=True)
    pos1_ref[...] = jnp.sum((offsets + c1tot + c2ex) * oh2, axis=1,
                            keepdims=True)


def _shared_body(x_ref, sg_ref, su_ref, sd_ref, out_ref):
    g = lax.dot_general(x_ref[...], sg_ref[...], (((1,), (1,)), ((), ())),
                        preferred_element_type=jnp.float32)
    u = lax.dot_general(x_ref[...], su_ref[...], (((1,), (1,)), ((), ())),
                        preferred_element_type=jnp.float32)
    h = jax.nn.silu(g) * u
    out_ref[...] = lax.dot_general(h, sd_ref[...], (((1,), (1,)), ((), ())),
                                   preferred_element_type=jnp.float32)


def _grouped_body(tile_s, exp_s, lo_s, hi_s, x_ref, wg_ref, wu_ref, wd_ref,
                  y_ref):
    i = pl.program_id(0)
    lo = lo_s[i]
    hi = hi_s[i]
    tile = tile_s[i]

    @pl.when(hi > lo)
    def _work():
        xb = x_ref[...]
        g = lax.dot_general(xb, wg_ref[0], (((1,), (1,)), ((), ())),
                            preferred_element_type=jnp.float32)
        u = lax.dot_general(xb, wu_ref[0], (((1,), (1,)), ((), ())),
                            preferred_element_type=jnp.float32)
        h = jax.nn.silu(g) * u
        y = lax.dot_general(h, wd_ref[0], (((1,), (1,)), ((), ())),
                            preferred_element_type=jnp.float32)
        rows = tile * T + lax.broadcasted_iota(jnp.int32, (T, 1), 0)
        m = ((rows >= lo) & (rows < hi)).astype(jnp.float32)
        contrib = y * m

        @pl.when(lo == tile * T)
        def _init():
            y_ref[...] = contrib

        @pl.when(lo != tile * T)
        def _acc():
            y_ref[...] += contrib


def _mesh():
    return plsc.VectorSubcoreMesh(core_axis_name="c", subcore_axis_name="s",
                                  num_cores=NC, num_subcores=NS)


@functools.cache
def _build_sc_dispatch():
    return functools.partial(
        pl.kernel,
        out_type=jax.ShapeDtypeStruct((P, H), jnp.float32),
        mesh=_mesh(),
        scratch_types=[
            pltpu.VMEM((16,), jnp.int32),
            pltpu.VMEM((16,), jnp.int32),
            pltpu.VMEM((16, H), jnp.float32),
            pltpu.VMEM((16, H), jnp.float32),
            pltpu.SemaphoreType.DMA,
            pltpu.SemaphoreType.DMA,
            pltpu.SemaphoreType.DMA,
            pltpu.SemaphoreType.DMA,
        ],
    )(_sc_dispatch_body)


def _sc_dispatch(xf, pos):
    return _build_sc_dispatch()(xf, pos)


def _sc_dispatch_body(x_hbm, pos_hbm, xs_hbm, idx_a, idx_b, row_a, row_b,
                      sem_ra, sem_rb, sem_wa, sem_wb):
    wid = lax.axis_index("s") * NC + lax.axis_index("c")
    k = wid // 16
    tb = (wid % 16) * 128
    idx = [idx_a, idx_b]
    row = [row_a, row_b]
    sem_r = [sem_ra, sem_rb]
    sem_w = [sem_wa, sem_wb]
    reads = [None, None]
    writes = [None, None]
    nch = 8
    pltpu.sync_copy(pos_hbm.at[k, pl.ds(tb, 16)], idx[0])
    reads[0] = pltpu.async_copy(x_hbm.at[pl.ds(tb, 16)], row[0], sem_r[0])
    for c in range(nch):
        s = c % 2
        o = 1 - s
        if c + 1 < nch:
            if writes[o] is not None:
                writes[o].wait()
                writes[o] = None
            base = tb + 16 * (c + 1)
            pltpu.sync_copy(pos_hbm.at[k, pl.ds(base, 16)], idx[o])
            reads[o] = pltpu.async_copy(x_hbm.at[pl.ds(base, 16)], row[o],
                                        sem_r[o])
        reads[s].wait()
        writes[s] = pltpu.async_copy(row[s], xs_hbm.at[idx[s]], sem_w[s])
    writes[0].wait()
    writes[1].wait()


_CT = 8
_NCH = 64 // _CT


@functools.cache
def _build_sc_combine():
    return functools.partial(
        pl.kernel,
        out_type=jax.ShapeDtypeStruct((S, H), jnp.float32),
        mesh=_mesh(),
        scratch_types=[
            pltpu.VMEM((2 * _CT,), jnp.int32),
            pltpu.VMEM((2 * _CT,), jnp.int32),
            pltpu.VMEM((2 * _CT, 16), jnp.float32),
            pltpu.VMEM((2 * _CT, 16), jnp.float32),
            pltpu.VMEM((2 * _CT, H), jnp.float32),
            pltpu.VMEM((2 * _CT, H), jnp.float32),
            pltpu.VMEM((_CT, H), jnp.float32),
            pltpu.VMEM((_CT, H), jnp.float32),
            pltpu.SemaphoreType.DMA,
            pltpu.SemaphoreType.DMA,
            pltpu.SemaphoreType.DMA,
            pltpu.SemaphoreType.DMA,
        ],
    )(_sc_combine_body)


def _sc_combine(shared, y, pos, pw):
    return _build_sc_combine()(shared, y, pos, pw)


def _sc_combine_body(sh_hbm, y_hbm, pos_hbm, pw_hbm, out_hbm,
                     idx_a, idx_b, p_a, p_b, y_a, y_b, s_a, s_b,
                     sem_ga, sem_gb, sem_wa, sem_wb):
    wid = lax.axis_index("s") * NC + lax.axis_index("c")
    idx = [idx_a, idx_b]
    pb = [p_a, p_b]
    yb = [y_a, y_b]
    sb = [s_a, s_b]
    sem_g = [sem_ga, sem_gb]
    sem_w = [sem_wa, sem_wb]
    gets = [None, None]
    puts = [None, None]

    def start(c, s):
        base = wid * 64 + _CT * c
        pltpu.sync_copy(pos_hbm.at[0, pl.ds(base, _CT)],
                        idx[s].at[pl.ds(0, _CT)])
        pltpu.sync_copy(pos_hbm.at[1, pl.ds(base, _CT)],
                        idx[s].at[pl.ds(_CT, _CT)])
        pltpu.sync_copy(pw_hbm.at[0, pl.ds(base, _CT)],
                        pb[s].at[pl.ds(0, _CT)])
        pltpu.sync_copy(pw_hbm.at[1, pl.ds(base, _CT)],
                        pb[s].at[pl.ds(_CT, _CT)])
        gets[s] = (pltpu.async_copy(y_hbm.at[idx[s]], yb[s], sem_g[s]),
                   pltpu.async_copy(sh_hbm.at[pl.ds(base, _CT)], sb[s],
                                    sem_g[s]))

    start(0, 0)
    for c in range(_NCH):
        s = c % 2
        o = 1 - s
        if c + 1 < _NCH:
            if puts[o] is not None:
                puts[o].wait()
                puts[o] = None
            start(c + 1, o)
        gets[s][0].wait()
        gets[s][1].wait()

        def tok(t, _):
            p0b = pb[s][t, :]
            p1b = pb[s][_CT + t, :]

            def col(j, _):
                d = pl.ds(j * 16, 16)
                sb[s][t, d] = (sb[s][t, d] + p0b * yb[s][t, d]
                               + p1b * yb[s][_CT + t, d])
                return 0

            lax.fori_loop(0, H // 16, col, 0, unroll=8)
            return 0

        lax.fori_loop(0, _CT, tok, 0)
        base = wid * 64 + _CT * c
        puts[s] = pltpu.async_copy(sb[s], out_hbm.at[pl.ds(base, _CT)],
                                   sem_w[s])
    puts[0].wait()
    puts[1].wait()


def _make_schedule(counts):
    counts_i = counts[0].astype(jnp.int32)
    offs = jnp.concatenate(
        [jnp.zeros((1,), jnp.int32), jnp.cumsum(counts_i)])
    bp = jnp.sort(jnp.concatenate(
        [jnp.arange(NT, dtype=jnp.int32) * T, offs[1:E]]))
    nxt = jnp.concatenate([bp[1:], jnp.array([P], jnp.int32)])
    item_tile = jnp.clip(bp // T, 0, NT - 1)
    item_expert = jnp.clip(
        jnp.searchsorted(offs, bp, side="right") - 1, 0, E - 1
    ).astype(jnp.int32)
    return item_tile, item_expert, bp, nxt


@jax.jit
def kernel(x, gate_w, e_bias, wg, wu, wd, sg, su, sd):
    xf = x.reshape(S, H)

    pos0, pos1, p0, p1, counts, mv = pl.pallas_call(
        _routing_body,
        out_shape=(
            jax.ShapeDtypeStruct((S, 1), jnp.float32),
            jax.ShapeDtypeStruct((S, 1), jnp.float32),
            jax.ShapeDtypeStruct((S, 1), jnp.float32),
            jax.ShapeDtypeStruct((S, 1), jnp.float32),
            jax.ShapeDtypeStruct((1, E), jnp.float32),
            jax.ShapeDtypeStruct((1, 1), jnp.float32),
        ),
    )(xf, gate_w, e_bias.reshape(1, E))

    pos = jnp.stack([pos0[:, 0], pos1[:, 0]]).astype(jnp.int32)
    pw = jnp.broadcast_to(
        jnp.stack([p0[:, 0], p1[:, 0]])[:, :, None], (K, S, 16))
    item_tile, item_expert, item_lo, item_hi = _make_schedule(counts)

    x_sorted = _sc_dispatch(xf, pos)

    shared = pl.pallas_call(
        _shared_body,
        grid=(8,),
        in_specs=[
            pl.BlockSpec((S // 8, H), lambda t: (t, 0)),
            pl.BlockSpec((I, H), lambda t: (0, 0)),
            pl.BlockSpec((I, H), lambda t: (0, 0)),
            pl.BlockSpec((H, I), lambda t: (0, 0)),
        ],
        out_specs=pl.BlockSpec((S // 8, H), lambda t: (t, 0)),
        out_shape=jax.ShapeDtypeStruct((S, H), jnp.float32),
    )(xf, sg, su, sd)

    y_sorted = pl.pallas_call(
        _grouped_body,
        grid_spec=pltpu.PrefetchScalarGridSpec(
            num_scalar_prefetch=4,
            grid=(NITEMS,),
            in_specs=[
                pl.BlockSpec((T, H), lambda i, ts, es, ls, hs: (ts[i], 0)),
                pl.BlockSpec((1, I, H),
                             lambda i, ts, es, ls, hs: (es[i], 0, 0)),
                pl.BlockSpec((1, I, H),
                             lambda i, ts, es, ls, hs: (es[i], 0, 0)),
                pl.BlockSpec((1, H, I),
                             lambda i, ts, es, ls, hs: (es[i], 0, 0)),
            ],
            out_specs=pl.BlockSpec((T, H),
                                   lambda i, ts, es, ls, hs: (ts[i], 0)),
        ),
        out_shape=jax.ShapeDtypeStruct((P, H), jnp.float32),
    )(item_tile, item_expert, item_lo, item_hi, x_sorted, wg, wu, wd)

    out = _sc_combine(shared, y_sorted, pos, pw)

    return (out.reshape(B, S, H), jnp.float32(0.0), mv[0, 0])


# speedup vs baseline: 2.3821x; 1.3075x over previous
import functools

import jax
import jax.numpy as jnp
from jax import lax
from jax.experimental import pallas as pl
from jax.experimental.pallas import tpu as pltpu
from jax.experimental.pallas import tpu_sc as plsc

B, S, H = 1, 2048, 2048
I = 1024
E = 8
K = 2
P = S * K
T = 256
NT = P // T
NITEMS = NT + E - 1

NC, NS = 2, 16
NW = NC * NS


def _routing_body(x_ref, gw_ref, bias_ref, pos0_ref, pos1_ref, p0_ref, p1_ref,
                  counts_ref, mv_ref):
    xf = x_ref[...]
    scores = lax.dot_general(
        xf, gw_ref[...], (((1,), (1,)), ((), ())),
        preferred_element_type=jnp.float32)
    biased = scores + bias_ref[...]
    iota = lax.broadcasted_iota(jnp.int32, (S, E), 1)
    neg_inf = jnp.float32(-jnp.inf)

    v1 = jnp.max(biased, axis=1, keepdims=True)
    idx1 = jnp.min(jnp.where(biased == v1, iota, E), axis=1, keepdims=True)
    masked = jnp.where(iota == idx1, neg_inf, biased)
    v2 = jnp.max(masked, axis=1, keepdims=True)
    idx2 = jnp.min(jnp.where(masked == v2, iota, E), axis=1, keepdims=True)

    u1 = jnp.max(scores, axis=1, keepdims=True)
    uidx1 = jnp.min(jnp.where(scores == u1, iota, E), axis=1, keepdims=True)
    u2 = jnp.max(jnp.where(iota == uidx1, neg_inf, scores), axis=1,
                 keepdims=True)
    p1 = jax.nn.sigmoid(u1)
    p2 = jax.nn.sigmoid(u2)
    ps = p1 + p2
    p0_ref[...] = p1 / ps
    p1_ref[...] = p2 / ps

    oh1 = (iota == idx1).astype(jnp.float32)
    oh2 = (iota == idx2).astype(jnp.float32)

    counts = jnp.sum(oh1 + oh2, axis=0, keepdims=True)
    counts_ref[...] = counts
    freq = counts / jnp.float32(P)
    fmean = jnp.sum(freq) / jnp.float32(E)
    mv_ref[...] = jnp.full((1, 1), (jnp.max(freq) - fmean) / fmean,
                           jnp.float32)

    tri = (lax.broadcasted_iota(jnp.int32, (S, S), 1)
           < lax.broadcasted_iota(jnp.int32, (S, S), 0)).astype(jnp.bfloat16)
    c1ex = lax.dot_general(tri, oh1.astype(jnp.bfloat16),
                           (((1,), (0,)), ((), ())),
                           preferred_element_type=jnp.float32)
    c2ex = lax.dot_general(tri, oh2.astype(jnp.bfloat16),
                           (((1,), (0,)), ((), ())),
                           preferred_element_type=jnp.float32)
    lt1 = (idx1 < iota).astype(jnp.float32)
    lt2 = (idx2 < iota).astype(jnp.float32)
    offsets = jnp.sum(lt1 + lt2, axis=0, keepdims=True)
    c1tot = jnp.sum(oh1, axis=0, keepdims=True)
    pos0_ref[...] = jnp.sum((offsets + c1ex) * oh1, axis=1, keepdims=True)
    pos1_ref[...] = jnp.sum((offsets + c1tot + c2ex) * oh2, axis=1,
                            keepdims=True)


def _shared_body(x_ref, sg_ref, su_ref, sd_ref, out_ref):
    g = lax.dot_general(x_ref[...], sg_ref[...], (((1,), (1,)), ((), ())),
                        preferred_element_type=jnp.float32)
    u = lax.dot_general(x_ref[...], su_ref[...], (((1,), (1,)), ((), ())),
                        preferred_element_type=jnp.float32)
    h = jax.nn.silu(g) * u
    out_ref[...] = lax.dot_general(h, sd_ref[...], (((1,), (1,)), ((), ())),
                                   preferred_element_type=jnp.float32)


def _grouped_body(tile_s, exp_s, lo_s, hi_s, x_ref, wg_ref, wu_ref, wd_ref,
                  y_ref):
    i = pl.program_id(0)
    lo = lo_s[i]
    hi = hi_s[i]
    tile = tile_s[i]

    @pl.when(hi > lo)
    def _work():
        xb = x_ref[...]
        g = lax.dot_general(xb, wg_ref[0], (((1,), (1,)), ((), ())),
                            preferred_element_type=jnp.float32)
        u = lax.dot_general(xb, wu_ref[0], (((1,), (1,)), ((), ())),
                            preferred_element_type=jnp.float32)
        h = jax.nn.silu(g) * u
        y = lax.dot_general(h, wd_ref[0], (((1,), (1,)), ((), ())),
                            preferred_element_type=jnp.float32)
        rows = tile * T + lax.broadcasted_iota(jnp.int32, (T, 1), 0)
        m = ((rows >= lo) & (rows < hi)).astype(jnp.float32)
        contrib = y * m

        @pl.when(lo == tile * T)
        def _init():
            y_ref[...] = contrib

        @pl.when(lo != tile * T)
        def _acc():
            y_ref[...] += contrib


def _mesh():
    return plsc.VectorSubcoreMesh(core_axis_name="c", subcore_axis_name="s",
                                  num_cores=NC, num_subcores=NS)


@functools.cache
def _build_sc_dispatch():
    return functools.partial(
        pl.kernel,
        out_type=jax.ShapeDtypeStruct((P, H), jnp.float32),
        mesh=_mesh(),
        scratch_types=[
            pltpu.VMEM((16,), jnp.int32),
            pltpu.VMEM((16,), jnp.int32),
            pltpu.VMEM((16, H), jnp.float32),
            pltpu.VMEM((16, H), jnp.float32),
            pltpu.SemaphoreType.DMA,
            pltpu.SemaphoreType.DMA,
            pltpu.SemaphoreType.DMA,
            pltpu.SemaphoreType.DMA,
        ],
    )(_sc_dispatch_body)


def _sc_dispatch(xf, pos):
    return _build_sc_dispatch()(xf, pos)


def _sc_dispatch_body(x_hbm, pos_hbm, xs_hbm, idx_a, idx_b, row_a, row_b,
                      sem_ra, sem_rb, sem_wa, sem_wb):
    wid = lax.axis_index("s") * NC + lax.axis_index("c")
    k = wid // 16
    tb = (wid % 16) * 128
    idx = [idx_a, idx_b]
    row = [row_a, row_b]
    sem_r = [sem_ra, sem_rb]
    sem_w = [sem_wa, sem_wb]
    reads = [None, None]
    writes = [None, None]
    nch = 8
    pltpu.sync_copy(pos_hbm.at[k, pl.ds(tb, 16)], idx[0])
    reads[0] = pltpu.async_copy(x_hbm.at[pl.ds(tb, 16)], row[0], sem_r[0])
    for c in range(nch):
        s = c % 2
        o = 1 - s
        if c + 1 < nch:
            if writes[o] is not None:
                writes[o].wait()
                writes[o] = None
            base = tb + 16 * (c + 1)
            pltpu.sync_copy(pos_hbm.at[k, pl.ds(base, 16)], idx[o])
            reads[o] = pltpu.async_copy(x_hbm.at[pl.ds(base, 16)], row[o],
                                        sem_r[o])
        reads[s].wait()
        writes[s] = pltpu.async_copy(row[s], xs_hbm.at[idx[s]], sem_w[s])
    writes[0].wait()
    writes[1].wait()


_CT = 8
_NCH = 64 // _CT


@functools.cache
def _build_sc_combine():
    return functools.partial(
        pl.kernel,
        out_type=jax.ShapeDtypeStruct((S, H), jnp.float32),
        mesh=_mesh(),
        scratch_types=[
            pltpu.VMEM((2 * _CT,), jnp.int32),
            pltpu.VMEM((2 * _CT,), jnp.int32),
            pltpu.VMEM((2 * _CT, 16), jnp.float32),
            pltpu.VMEM((2 * _CT, 16), jnp.float32),
            pltpu.VMEM((2 * _CT, H), jnp.float32),
            pltpu.VMEM((2 * _CT, H), jnp.float32),
            pltpu.VMEM((_CT, H), jnp.float32),
            pltpu.VMEM((_CT, H), jnp.float32),
            pltpu.SemaphoreType.DMA,
            pltpu.SemaphoreType.DMA,
            pltpu.SemaphoreType.DMA,
            pltpu.SemaphoreType.DMA,
        ],
    )(_sc_combine_body)


def _sc_combine(shared, y, pos, pw):
    return _build_sc_combine()(shared, y, pos, pw)


def _sc_combine_body(sh_hbm, y_hbm, pos_hbm, pw_hbm, out_hbm,
                     idx_a, idx_b, p_a, p_b, y_a, y_b, s_a, s_b,
                     sem_ga, sem_gb, sem_wa, sem_wb):
    wid = lax.axis_index("s") * NC + lax.axis_index("c")
    idx = [idx_a, idx_b]
    pb = [p_a, p_b]
    yb = [y_a, y_b]
    sb = [s_a, s_b]
    sem_g = [sem_ga, sem_gb]
    sem_w = [sem_wa, sem_wb]
    gets = [None, None]
    puts = [None, None]

    def start(c, s):
        base = wid * 64 + _CT * c
        pltpu.sync_copy(pos_hbm.at[0, pl.ds(base, _CT)],
                        idx[s].at[pl.ds(0, _CT)])
        pltpu.sync_copy(pos_hbm.at[1, pl.ds(base, _CT)],
                        idx[s].at[pl.ds(_CT, _CT)])
        pltpu.sync_copy(pw_hbm.at[0, pl.ds(base, _CT)],
                        pb[s].at[pl.ds(0, _CT)])
        pltpu.sync_copy(pw_hbm.at[1, pl.ds(base, _CT)],
                        pb[s].at[pl.ds(_CT, _CT)])
        gets[s] = (pltpu.async_copy(y_hbm.at[idx[s]], yb[s], sem_g[s]),
                   pltpu.async_copy(sh_hbm.at[pl.ds(base, _CT)], sb[s],
                                    sem_g[s]))

    start(0, 0)
    for c in range(_NCH):
        s = c % 2
        o = 1 - s
        if c + 1 < _NCH:
            if puts[o] is not None:
                puts[o].wait()
                puts[o] = None
            start(c + 1, o)
        gets[s][0].wait()
        gets[s][1].wait()

        def tok(t, _):
            p0b = pb[s][t, :]
            p1b = pb[s][_CT + t, :]

            def col(j, _):
                d = pl.ds(j * 16, 16)
                sb[s][t, d] = (sb[s][t, d] + p0b * yb[s][t, d]
                               + p1b * yb[s][_CT + t, d])
                return 0

            lax.fori_loop(0, H // 16, col, 0, unroll=8)
            return 0

        lax.fori_loop(0, _CT, tok, 0)
        base = wid * 64 + _CT * c
        puts[s] = pltpu.async_copy(sb[s], out_hbm.at[pl.ds(base, _CT)],
                                   sem_w[s])
    puts[0].wait()
    puts[1].wait()


def _make_schedule(counts):
    counts_i = counts[0].astype(jnp.int32)
    offs = jnp.concatenate(
        [jnp.zeros((1,), jnp.int32), jnp.cumsum(counts_i)])
    bp = jnp.sort(jnp.concatenate(
        [jnp.arange(NT, dtype=jnp.int32) * T, offs[1:E]]))
    nxt = jnp.concatenate([bp[1:], jnp.array([P], jnp.int32)])
    item_tile = jnp.clip(bp // T, 0, NT - 1)
    item_expert = jnp.clip(
        jnp.searchsorted(offs, bp, side="right") - 1, 0, E - 1
    ).astype(jnp.int32)
    return item_tile, item_expert, bp, nxt


@jax.jit
def kernel(x, gate_w, e_bias, wg, wu, wd, sg, su, sd):
    xf = x.reshape(S, H)

    pos0, pos1, p0, p1, counts, mv = pl.pallas_call(
        _routing_body,
        out_shape=(
            jax.ShapeDtypeStruct((S, 1), jnp.float32),
            jax.ShapeDtypeStruct((S, 1), jnp.float32),
            jax.ShapeDtypeStruct((S, 1), jnp.float32),
            jax.ShapeDtypeStruct((S, 1), jnp.float32),
            jax.ShapeDtypeStruct((1, E), jnp.float32),
            jax.ShapeDtypeStruct((1, 1), jnp.float32),
        ),
    )(xf, gate_w, e_bias.reshape(1, E))

    pos = jnp.stack([pos0[:, 0], pos1[:, 0]]).astype(jnp.int32)
    pw = jnp.broadcast_to(
        jnp.stack([p0[:, 0], p1[:, 0]])[:, :, None], (K, S, 16))
    item_tile, item_expert, item_lo, item_hi = _make_schedule(counts)

    x_sorted = _sc_dispatch(xf, pos)

    shared = pl.pallas_call(
        _shared_body,
        grid=(8,),
        in_specs=[
            pl.BlockSpec((S // 8, H), lambda t: (t, 0)),
            pl.BlockSpec((I, H), lambda t: (0, 0)),
            pl.BlockSpec((I, H), lambda t: (0, 0)),
            pl.BlockSpec((H, I), lambda t: (0, 0)),
        ],
        out_specs=pl.BlockSpec((S // 8, H), lambda t: (t, 0)),
        out_shape=jax.ShapeDtypeStruct((S, H), jnp.float32),
    )(xf, sg, su, sd)

    y_sorted = pl.pallas_call(
        _grouped_body,
        grid_spec=pltpu.PrefetchScalarGridSpec(
            num_scalar_prefetch=4,
            grid=(NITEMS,),
            in_specs=[
                pl.BlockSpec((T, H), lambda i, ts, es, ls, hs: (ts[i], 0)),
                pl.BlockSpec((1, I, H),
                             lambda i, ts, es, ls, hs: (es[i], 0, 0)),
                pl.BlockSpec((1, I, H),
                             lambda i, ts, es, ls, hs: (es[i], 0, 0)),
                pl.BlockSpec((1, H, I),
                             lambda i, ts, es, ls, hs: (es[i], 0, 0)),
            ],
            out_specs=pl.BlockSpec((T, H),
                                   lambda i, ts, es, ls, hs: (ts[i], 0)),
        ),
        out_shape=jax.ShapeDtypeStruct((P, H), jnp.float32),
    )(item_tile, item_expert, item_lo, item_hi, x_sorted, wg, wu, wd)

    out = _sc_combine(shared, y_sorted, pos, pw)

    return (out.reshape(B, S, H), jnp.float32(0.0), mv[0, 0])


# speedup vs baseline: 2.4342x; 1.0219x over previous
import functools

import jax
import jax.numpy as jnp
from jax import lax
from jax.experimental import pallas as pl
from jax.experimental.pallas import tpu as pltpu
from jax.experimental.pallas import tpu_sc as plsc

B, S, H = 1, 2048, 2048
I = 1024
E = 8
K = 2
P = S * K
T = 256
NT = P // T
NITEMS = NT + E - 1

NC, NS = 2, 16
NW = NC * NS


def _routing_body(x_ref, gw_ref, bias_ref, pos0_ref, pos1_ref, p0_ref, p1_ref,
                  counts_ref, mv_ref, tile_ref, exp_ref, lo_ref, hi_ref):
    xf = x_ref[...]
    scores = lax.dot_general(
        xf, gw_ref[...], (((1,), (1,)), ((), ())),
        preferred_element_type=jnp.float32)
    biased = scores + bias_ref[...]
    iota = lax.broadcasted_iota(jnp.int32, (S, E), 1)
    neg_inf = jnp.float32(-jnp.inf)

    v1 = jnp.max(biased, axis=1, keepdims=True)
    idx1 = jnp.min(jnp.where(biased == v1, iota, E), axis=1, keepdims=True)
    masked = jnp.where(iota == idx1, neg_inf, biased)
    v2 = jnp.max(masked, axis=1, keepdims=True)
    idx2 = jnp.min(jnp.where(masked == v2, iota, E), axis=1, keepdims=True)

    u1 = jnp.max(scores, axis=1, keepdims=True)
    uidx1 = jnp.min(jnp.where(scores == u1, iota, E), axis=1, keepdims=True)
    u2 = jnp.max(jnp.where(iota == uidx1, neg_inf, scores), axis=1,
                 keepdims=True)
    p1 = jax.nn.sigmoid(u1)
    p2 = jax.nn.sigmoid(u2)
    ps = p1 + p2
    p0_ref[...] = p1 / ps
    p1_ref[...] = p2 / ps

    oh1 = (iota == idx1).astype(jnp.float32)
    oh2 = (iota == idx2).astype(jnp.float32)

    counts = jnp.sum(oh1 + oh2, axis=0, keepdims=True)
    counts_ref[...] = counts
    freq = counts / jnp.float32(P)
    fmean = jnp.sum(freq) / jnp.float32(E)
    mv_ref[...] = jnp.full((1, 1), (jnp.max(freq) - fmean) / fmean,
                           jnp.float32)

    tri = (lax.broadcasted_iota(jnp.int32, (S, S), 1)
           < lax.broadcasted_iota(jnp.int32, (S, S), 0)).astype(jnp.bfloat16)
    c1ex = lax.dot_general(tri, oh1.astype(jnp.bfloat16),
                           (((1,), (0,)), ((), ())),
                           preferred_element_type=jnp.float32)
    c2ex = lax.dot_general(tri, oh2.astype(jnp.bfloat16),
                           (((1,), (0,)), ((), ())),
                           preferred_element_type=jnp.float32)
    lt1 = (idx1 < iota).astype(jnp.float32)
    lt2 = (idx2 < iota).astype(jnp.float32)
    offsets = jnp.sum(lt1 + lt2, axis=0, keepdims=True)
    c1tot = jnp.sum(oh1, axis=0, keepdims=True)
    pos0_ref[...] = jnp.sum((offsets + c1ex) * oh1, axis=1, keepdims=True)
    pos1_ref[...] = jnp.sum((offsets + c1tot + c2ex) * oh2, axis=1,
                            keepdims=True)

    ones_col = jnp.ones((S, 1), jnp.bfloat16)
    counts_col = lax.dot_general((oh1 + oh2).astype(jnp.bfloat16), ones_col,
                                 (((0,), (0,)), ((), ())),
                                 preferred_element_type=jnp.float32)
    offsets_col = lax.dot_general((lt1 + lt2).astype(jnp.bfloat16), ones_col,
                                  (((0,), (0,)), ((), ())),
                                  preferred_element_type=jnp.float32)
    iota_nt_col = lax.broadcasted_iota(jnp.int32, (NT, 1), 0).astype(
        jnp.float32)
    tstart_col = iota_nt_col * T
    off_int_row = offsets[:, 1:E]
    off_int_col = offsets_col[1:E, :]
    rank_tile = (iota_nt_col
                 + jnp.sum((off_int_row <= tstart_col).astype(jnp.float32),
                           axis=1, keepdims=True))
    rank_off = (lax.broadcasted_iota(jnp.int32, (E - 1, 1), 0).astype(
                    jnp.float32)
                + jnp.floor((off_int_col + (T - 1)) * (1.0 / T)))
    i_row = lax.broadcasted_iota(jnp.int32, (1, NITEMS), 1).astype(
        jnp.float32)
    bp = (jnp.sum((rank_tile == i_row).astype(jnp.float32) * tstart_col,
                  axis=0, keepdims=True)
          + jnp.sum((rank_off == i_row).astype(jnp.float32) * off_int_col,
                    axis=0, keepdims=True))
    nxt = (jnp.sum((rank_tile == i_row + 1).astype(jnp.float32) * tstart_col,
                   axis=0, keepdims=True)
           + jnp.sum((rank_off == i_row + 1).astype(jnp.float32)
                     * off_int_col, axis=0, keepdims=True)
           + jnp.float32(P) * (i_row == NITEMS - 1).astype(jnp.float32))
    cum_inc_col = offsets_col + counts_col
    expert = jnp.clip(
        jnp.sum((cum_inc_col <= bp).astype(jnp.float32), axis=0,
                keepdims=True), 0, E - 1)
    tile_ref[...] = jnp.clip(bp * (1.0 / T), 0, NT - 1).astype(jnp.int32)
    exp_ref[...] = expert.astype(jnp.int32)
    lo_ref[...] = bp.astype(jnp.int32)
    hi_ref[...] = nxt.astype(jnp.int32)


def _shared_body(x_ref, sg_ref, su_ref, sd_ref, out_ref):
    g = lax.dot_general(x_ref[...], sg_ref[...], (((1,), (1,)), ((), ())),
                        preferred_element_type=jnp.float32)
    u = lax.dot_general(x_ref[...], su_ref[...], (((1,), (1,)), ((), ())),
                        preferred_element_type=jnp.float32)
    h = jax.nn.silu(g) * u
    out_ref[...] = lax.dot_general(h, sd_ref[...], (((1,), (1,)), ((), ())),
                                   preferred_element_type=jnp.float32)


def _grouped_body(tile_s, exp_s, lo_s, hi_s, x_ref, wg_ref, wu_ref, wd_ref,
                  y_ref):
    i = pl.program_id(0)
    lo = lo_s[i]
    hi = hi_s[i]
    tile = tile_s[i]

    @pl.when(hi > lo)
    def _work():
        xb = x_ref[...]
        g = lax.dot_general(xb, wg_ref[0], (((1,), (1,)), ((), ())),
                            preferred_element_type=jnp.float32)
        u = lax.dot_general(xb, wu_ref[0], (((1,), (1,)), ((), ())),
                            preferred_element_type=jnp.float32)
        h = jax.nn.silu(g) * u
        y = lax.dot_general(h, wd_ref[0], (((1,), (1,)), ((), ())),
                            preferred_element_type=jnp.float32)
        rows = tile * T + lax.broadcasted_iota(jnp.int32, (T, 1), 0)
        m = ((rows >= lo) & (rows < hi)).astype(jnp.float32)
        contrib = y * m

        @pl.when(lo == tile * T)
        def _init():
            y_ref[...] = contrib

        @pl.when(lo != tile * T)
        def _acc():
            y_ref[...] += contrib


def _mesh():
    return plsc.VectorSubcoreMesh(core_axis_name="c", subcore_axis_name="s",
                                  num_cores=NC, num_subcores=NS)


@functools.cache
def _build_sc_dispatch():
    return functools.partial(
        pl.kernel,
        out_type=jax.ShapeDtypeStruct((P, H), jnp.float32),
        mesh=_mesh(),
        scratch_types=[
            pltpu.VMEM((16,), jnp.int32),
            pltpu.VMEM((16,), jnp.int32),
            pltpu.VMEM((16, H), jnp.float32),
            pltpu.VMEM((16, H), jnp.float32),
            pltpu.SemaphoreType.DMA,
            pltpu.SemaphoreType.DMA,
            pltpu.SemaphoreType.DMA,
            pltpu.SemaphoreType.DMA,
        ],
    )(_sc_dispatch_body)


def _sc_dispatch(xf, pos):
    return _build_sc_dispatch()(xf, pos)


def _sc_dispatch_body(x_hbm, pos_hbm, xs_hbm, idx_a, idx_b, row_a, row_b,
                      sem_ra, sem_rb, sem_wa, sem_wb):
    wid = lax.axis_index("s") * NC + lax.axis_index("c")
    k = wid // 16
    tb = (wid % 16) * 128
    idx = [idx_a, idx_b]
    row = [row_a, row_b]
    sem_r = [sem_ra, sem_rb]
    sem_w = [sem_wa, sem_wb]
    reads = [None, None]
    writes = [None, None]
    nch = 8
    pltpu.sync_copy(pos_hbm.at[k, pl.ds(tb, 16)], idx[0])
    reads[0] = pltpu.async_copy(x_hbm.at[pl.ds(tb, 16)], row[0], sem_r[0])
    for c in range(nch):
        s = c % 2
        o = 1 - s
        if c + 1 < nch:
            if writes[o] is not None:
                writes[o].wait()
                writes[o] = None
            base = tb + 16 * (c + 1)
            pltpu.sync_copy(pos_hbm.at[k, pl.ds(base, 16)], idx[o])
            reads[o] = pltpu.async_copy(x_hbm.at[pl.ds(base, 16)], row[o],
                                        sem_r[o])
        reads[s].wait()
        writes[s] = pltpu.async_copy(row[s], xs_hbm.at[idx[s]], sem_w[s])
    writes[0].wait()
    writes[1].wait()


_CT = 8
_NCH = 64 // _CT


@functools.cache
def _build_sc_combine():
    return functools.partial(
        pl.kernel,
        out_type=jax.ShapeDtypeStruct((S, H), jnp.float32),
        mesh=_mesh(),
        scratch_types=[
            pltpu.VMEM((2 * _CT,), jnp.int32),
            pltpu.VMEM((2 * _CT,), jnp.int32),
            pltpu.VMEM((2 * _CT, 16), jnp.float32),
            pltpu.VMEM((2 * _CT, 16), jnp.float32),
            pltpu.VMEM((2 * _CT, H), jnp.float32),
            pltpu.VMEM((2 * _CT, H), jnp.float32),
            pltpu.VMEM((_CT, H), jnp.float32),
            pltpu.VMEM((_CT, H), jnp.float32),
            pltpu.SemaphoreType.DMA,
            pltpu.SemaphoreType.DMA,
            pltpu.SemaphoreType.DMA,
            pltpu.SemaphoreType.DMA,
        ],
    )(_sc_combine_body)


def _sc_combine(shared, y, pos, pw):
    return _build_sc_combine()(shared, y, pos, pw)


def _sc_combine_body(sh_hbm, y_hbm, pos_hbm, pw_hbm, out_hbm,
                     idx_a, idx_b, p_a, p_b, y_a, y_b, s_a, s_b,
                     sem_ga, sem_gb, sem_wa, sem_wb):
    wid = lax.axis_index("s") * NC + lax.axis_index("c")
    idx = [idx_a, idx_b]
    pb = [p_a, p_b]
    yb = [y_a, y_b]
    sb = [s_a, s_b]
    sem_g = [sem_ga, sem_gb]
    sem_w = [sem_wa, sem_wb]
    gets = [None, None]
    puts = [None, None]

    def start(c, s):
        base = wid * 64 + _CT * c
        pltpu.sync_copy(pos_hbm.at[0, pl.ds(base, _CT)],
                        idx[s].at[pl.ds(0, _CT)])
        pltpu.sync_copy(pos_hbm.at[1, pl.ds(base, _CT)],
                        idx[s].at[pl.ds(_CT, _CT)])
        pltpu.sync_copy(pw_hbm.at[0, pl.ds(base, _CT)],
                        pb[s].at[pl.ds(0, _CT)])
        pltpu.sync_copy(pw_hbm.at[1, pl.ds(base, _CT)],
                        pb[s].at[pl.ds(_CT, _CT)])
        gets[s] = (pltpu.async_copy(y_hbm.at[idx[s]], yb[s], sem_g[s]),
                   pltpu.async_copy(sh_hbm.at[pl.ds(base, _CT)], sb[s],
                                    sem_g[s]))

    start(0, 0)
    for c in range(_NCH):
        s = c % 2
        o = 1 - s
        if c + 1 < _NCH:
            if puts[o] is not None:
                puts[o].wait()
                puts[o] = None
            start(c + 1, o)
        gets[s][0].wait()
        gets[s][1].wait()

        def tok(t, _):
            p0b = pb[s][t, :]
            p1b = pb[s][_CT + t, :]

            def col(j, _):
                d = pl.ds(j * 16, 16)
                sb[s][t, d] = (sb[s][t, d] + p0b * yb[s][t, d]
                               + p1b * yb[s][_CT + t, d])
                return 0

            lax.fori_loop(0, H // 16, col, 0, unroll=8)
            return 0

        lax.fori_loop(0, _CT, tok, 0)
        base = wid * 64 + _CT * c
        puts[s] = pltpu.async_copy(sb[s], out_hbm.at[pl.ds(base, _CT)],
                                   sem_w[s])
    puts[0].wait()
    puts[1].wait()


@jax.jit
def kernel(x, gate_w, e_bias, wg, wu, wd, sg, su, sd):
    xf = x.reshape(S, H)

    (pos0, pos1, p0, p1, counts, mv, item_tile, item_expert, item_lo,
     item_hi) = pl.pallas_call(
        _routing_body,
        out_shape=(
            jax.ShapeDtypeStruct((S, 1), jnp.float32),
            jax.ShapeDtypeStruct((S, 1), jnp.float32),
            jax.ShapeDtypeStruct((S, 1), jnp.float32),
            jax.ShapeDtypeStruct((S, 1), jnp.float32),
            jax.ShapeDtypeStruct((1, E), jnp.float32),
            jax.ShapeDtypeStruct((1, 1), jnp.float32),
            jax.ShapeDtypeStruct((1, NITEMS), jnp.int32),
            jax.ShapeDtypeStruct((1, NITEMS), jnp.int32),
            jax.ShapeDtypeStruct((1, NITEMS), jnp.int32),
            jax.ShapeDtypeStruct((1, NITEMS), jnp.int32),
        ),
    )(xf, gate_w, e_bias.reshape(1, E))

    pos = jnp.stack([pos0[:, 0], pos1[:, 0]]).astype(jnp.int32)
    pw = jnp.broadcast_to(
        jnp.stack([p0[:, 0], p1[:, 0]])[:, :, None], (K, S, 16))
    item_tile = item_tile.reshape(NITEMS)
    item_expert = item_expert.reshape(NITEMS)
    item_lo = item_lo.reshape(NITEMS)
    item_hi = item_hi.reshape(NITEMS)

    x_sorted = _sc_dispatch(xf, pos)

    shared = pl.pallas_call(
        _shared_body,
        grid=(8,),
        in_specs=[
            pl.BlockSpec((S // 8, H), lambda t: (t, 0)),
            pl.BlockSpec((I, H), lambda t: (0, 0)),
            pl.BlockSpec((I, H), lambda t: (0, 0)),
            pl.BlockSpec((H, I), lambda t: (0, 0)),
        ],
        out_specs=pl.BlockSpec((S // 8, H), lambda t: (t, 0)),
        out_shape=jax.ShapeDtypeStruct((S, H), jnp.float32),
    )(xf, sg, su, sd)

    y_sorted = pl.pallas_call(
        _grouped_body,
        grid_spec=pltpu.PrefetchScalarGridSpec(
            num_scalar_prefetch=4,
            grid=(NITEMS,),
            in_specs=[
                pl.BlockSpec((T, H), lambda i, ts, es, ls, hs: (ts[i], 0)),
                pl.BlockSpec((1, I, H),
                             lambda i, ts, es, ls, hs: (es[i], 0, 0)),
                pl.BlockSpec((1, I, H),
                             lambda i, ts, es, ls, hs: (es[i], 0, 0)),
                pl.BlockSpec((1, H, I),
                             lambda i, ts, es, ls, hs: (es[i], 0, 0)),
            ],
            out_specs=pl.BlockSpec((T, H),
                                   lambda i, ts, es, ls, hs: (ts[i], 0)),
        ),
        out_shape=jax.ShapeDtypeStruct((P, H), jnp.float32),
    )(item_tile, item_expert, item_lo, item_hi, x_sorted, wg, wu, wd)

    out = _sc_combine(shared, y_sorted, pos, pw)

    return (out.reshape(B, S, H), jnp.float32(0.0), mv[0, 0])
